# baseline (speedup 1.0000x reference)
import jax
import jax.numpy as jnp
from jax.experimental import pallas as pl
from jax.experimental.pallas import tpu as pltpu

V_COARSE = 4096
V_FINE = 16384
COARSE_DIM = 256
FINE_INPUT_DIM = 256
OUTPUT_DIM = 256
SCALAR_PROJ_DIM = 64

ROWS = 512


def _fused_body(x_coarse_ref, w_sym_ref, b_sym_ref, interp_ref, x_fine_ref,
                w_fuse1_ref, w_fuse2_ref, b_fuse_ref, out_ref, x_scalar_ref):
    @pl.when(pl.program_id(0) == 0)
    def _():
        x_scalar_ref[...] = (
            jnp.dot(x_coarse_ref[...], w_sym_ref[...],
                    preferred_element_type=jnp.float32)
            + b_sym_ref[...]
        )

    t = jnp.dot(interp_ref[...], x_scalar_ref[...],
                preferred_element_type=jnp.float32)
    out_ref[...] = (
        jnp.dot(t, w_fuse1_ref[...], preferred_element_type=jnp.float32)
        + jnp.dot(x_fine_ref[...], w_fuse2_ref[...],
                  preferred_element_type=jnp.float32)
        + b_fuse_ref[...]
    )


def kernel(x_coarse, x_fine_input, interp_matrix, W_sym, b_sym, W_fuse, b_fuse):
    w_fuse1 = W_fuse[:SCALAR_PROJ_DIM, :]
    w_fuse2 = W_fuse[SCALAR_PROJ_DIM:, :]
    b_sym2 = b_sym.reshape(1, SCALAR_PROJ_DIM)
    b_fuse2 = b_fuse.reshape(1, OUTPUT_DIM)

    grid = (V_FINE // ROWS,)
    return pl.pallas_call(
        _fused_body,
        grid=grid,
        in_specs=[
            pl.BlockSpec((V_COARSE, COARSE_DIM), lambda i: (0, 0)),
            pl.BlockSpec((COARSE_DIM, SCALAR_PROJ_DIM), lambda i: (0, 0)),
            pl.BlockSpec((1, SCALAR_PROJ_DIM), lambda i: (0, 0)),
            pl.BlockSpec((ROWS, V_COARSE), lambda i: (i, 0)),
            pl.BlockSpec((ROWS, FINE_INPUT_DIM), lambda i: (i, 0)),
            pl.BlockSpec((SCALAR_PROJ_DIM, OUTPUT_DIM), lambda i: (0, 0)),
            pl.BlockSpec((FINE_INPUT_DIM, OUTPUT_DIM), lambda i: (0, 0)),
            pl.BlockSpec((1, OUTPUT_DIM), lambda i: (0, 0)),
        ],
        out_specs=pl.BlockSpec((ROWS, OUTPUT_DIM), lambda i: (i, 0)),
        out_shape=jax.ShapeDtypeStruct((V_FINE, OUTPUT_DIM), jnp.float32),
        scratch_shapes=[pltpu.VMEM((V_COARSE, SCALAR_PROJ_DIM), jnp.float32)],
        compiler_params=pltpu.CompilerParams(
            dimension_semantics=("arbitrary",)),
    )(x_coarse, W_sym, b_sym2, interp_matrix, x_fine_input,
      w_fuse1, w_fuse2, b_fuse2)
